# Initial kernel scaffold; baseline (speedup 1.0000x reference)
import jax
import jax.numpy as jnp
from jax.experimental import pallas as pl


def kernel(x, pos_embed):
    raise NotImplementedError("write your pallas kernel here")


# speedup vs baseline: 8.0257x; 8.0257x over previous
import functools

import jax
import jax.numpy as jnp
from jax import lax
from jax.experimental import pallas as pl
from jax.experimental.pallas import tpu as pltpu
from jax.experimental.pallas import tpu_sc as plsc

_B = 16384
_S = 200
_D = 64
_NC = 2
_NS = 16
_NW = _NC * _NS
_PER_W = _B // _NW
_CHUNK = 4
_NDMA = _PER_W // _CHUNK


def _make_sc_broadcast():
    mesh = plsc.VectorSubcoreMesh(core_axis_name="c", subcore_axis_name="s")

    @functools.partial(
        pl.kernel,
        mesh=mesh,
        out_type=jax.ShapeDtypeStruct((_B, _S, _D), jnp.float32),
        scratch_types=[
            pltpu.VMEM((_CHUNK, _S, _D), jnp.float32),
            pltpu.SemaphoreType.DMA,
        ],
    )
    def body(pos_embed_hbm, out_hbm, buf, sem):
        wid = lax.axis_index("s") * _NC + lax.axis_index("c")
        base = wid * _PER_W
        for i in range(_CHUNK):
            pltpu.sync_copy(pos_embed_hbm.at[pl.ds(0, _S)], buf.at[i])
        copies = [
            pltpu.async_copy(
                buf, out_hbm.at[pl.ds(base + j * _CHUNK, _CHUNK)], sem
            )
            for j in range(_NDMA)
        ]
        for c in copies:
            c.wait()

    return body


_sc_broadcast = _make_sc_broadcast()


def kernel(x, pos_embed):
    del x
    return _sc_broadcast(pos_embed)
